# baseline (speedup 1.0000x reference)
import functools

import jax
import jax.numpy as jnp
from jax import lax
from jax.experimental import pallas as pl
from jax.experimental.pallas import tpu as pltpu, tpu_sc as plsc

N = 10000
D = 128
DH = D // 2
E = 320000

NPAD = 10240
ETOT = 330240
EPT = ETOT // 16
C = 480
NCHUNK = EPT // C
ROWS_PER_TILE = NPAD // 16

_BLK = 512


def _tc_in_body(x_ref, w_ref, asrc_ref, adst_ref, h_ref, es_ref, ed_ref):
    h = jnp.dot(x_ref[...], w_ref[...], preferred_element_type=jnp.float32)
    h_ref[0] = h[:, :DH]
    h_ref[1] = h[:, DH:]
    es_ref[...] = jnp.sum(h * asrc_ref[...], axis=-1, keepdims=True)
    ed_ref[...] = jnp.sum(h * adst_ref[...], axis=-1, keepdims=True)


def _tc_in(x_pad, w, asrc, adst):
    grid = (NPAD // _BLK,)
    return pl.pallas_call(
        _tc_in_body,
        grid=grid,
        in_specs=[
            pl.BlockSpec((_BLK, D), lambda i: (i, 0)),
            pl.BlockSpec((D, D), lambda i: (0, 0)),
            pl.BlockSpec((1, D), lambda i: (0, 0)),
            pl.BlockSpec((1, D), lambda i: (0, 0)),
        ],
        out_specs=[
            pl.BlockSpec((2, _BLK, DH), lambda i: (0, i, 0)),
            pl.BlockSpec((_BLK, 1), lambda i: (i, 0)),
            pl.BlockSpec((_BLK, 1), lambda i: (i, 0)),
        ],
        out_shape=[
            jax.ShapeDtypeStruct((2, NPAD, DH), jnp.float32),
            jax.ShapeDtypeStruct((NPAD, 1), jnp.float32),
            jax.ShapeDtypeStruct((NPAD, 1), jnp.float32),
        ],
    )(x_pad, w, asrc, adst)


def _tc_mid_body(u_ref, d_ref, w_ref, asrc_ref, adst_ref,
                 h_ref, es_ref, ed_ref):
    den = jnp.maximum(d_ref[...], 1e-30)
    u = jnp.concatenate([u_ref[0], u_ref[1]], axis=-1)
    g = jnp.maximum(u, 0.0) / den
    h = jnp.dot(g, w_ref[...], preferred_element_type=jnp.float32)
    h_ref[0] = h[:, :DH]
    h_ref[1] = h[:, DH:]
    es_ref[...] = jnp.sum(h * asrc_ref[...], axis=-1, keepdims=True)
    ed_ref[...] = jnp.sum(h * adst_ref[...], axis=-1, keepdims=True)


def _tc_mid(u, d, w, asrc, adst):
    grid = (NPAD // _BLK,)
    return pl.pallas_call(
        _tc_mid_body,
        grid=grid,
        in_specs=[
            pl.BlockSpec((2, _BLK, DH), lambda i: (0, i, 0)),
            pl.BlockSpec((_BLK, 1), lambda i: (i, 0)),
            pl.BlockSpec((D, D), lambda i: (0, 0)),
            pl.BlockSpec((1, D), lambda i: (0, 0)),
            pl.BlockSpec((1, D), lambda i: (0, 0)),
        ],
        out_specs=[
            pl.BlockSpec((2, _BLK, DH), lambda i: (0, i, 0)),
            pl.BlockSpec((_BLK, 1), lambda i: (i, 0)),
            pl.BlockSpec((_BLK, 1), lambda i: (i, 0)),
        ],
        out_shape=[
            jax.ShapeDtypeStruct((2, NPAD, DH), jnp.float32),
            jax.ShapeDtypeStruct((NPAD, 1), jnp.float32),
            jax.ShapeDtypeStruct((NPAD, 1), jnp.float32),
        ],
    )(u, d, w, asrc, adst)


_OBLK = 400


def _tc_out_body(u_ref, d_ref, o_ref):
    den = jnp.maximum(d_ref[...], 1e-30)
    g = jnp.concatenate([u_ref[0], u_ref[1]], axis=-1) / den
    m = jnp.max(g, axis=-1, keepdims=True)
    l = g - m
    o_ref[...] = l - jnp.log(jnp.sum(jnp.exp(l), axis=-1, keepdims=True))


def _tc_out(u, d):
    grid = (N // _OBLK,)
    return pl.pallas_call(
        _tc_out_body,
        grid=grid,
        in_specs=[
            pl.BlockSpec((2, _OBLK, DH), lambda i: (0, i, 0)),
            pl.BlockSpec((_OBLK, 1), lambda i: (i, 0)),
        ],
        out_specs=pl.BlockSpec((_OBLK, D), lambda i: (i, 0)),
        out_shape=jax.ShapeDtypeStruct((N, D), jnp.float32),
    )(u, d)


def _edge_body(h_hbm, es_hbm, ed_hbm, src_hbm, dst_hbm,
               u_hbm, den_hbm,
               es_l, ed_l, src_v, dst_v, rows_v, ee_v, zden,
               u_sh, den_sh,
               gsem):
    c = lax.axis_index("c")
    s = lax.axis_index("s")

    z16 = jnp.zeros((16,), jnp.float32)

    def _zrow(i, carry):
        for j in range(DH // 16):
            rows_v[i, pl.ds(j * 16, 16)] = z16
        return carry
    lax.fori_loop(0, C, _zrow, 0)
    for t in range(ROWS_PER_TILE // 16):
        zden[pl.ds(t * 16, 16)] = z16

    row0 = pl.multiple_of(s * ROWS_PER_TILE, 8)
    pltpu.sync_copy(rows_v, u_sh.at[pl.ds(row0, C)])
    pltpu.sync_copy(rows_v.at[pl.ds(0, ROWS_PER_TILE - C)],
                    u_sh.at[pl.ds(row0 + C, ROWS_PER_TILE - C)])
    pltpu.sync_copy(zden, den_sh.at[pl.ds(row0, ROWS_PER_TILE)])

    pltpu.sync_copy(es_hbm, es_l)
    pltpu.sync_copy(ed_hbm, ed_l)

    plsc.subcore_barrier()

    h_mine = h_hbm.at[c]

    def _chunk(k, carry):
        base = pl.multiple_of(s * EPT + k * C, 8)
        pltpu.sync_copy(src_hbm.at[pl.ds(base, C)], src_v)
        pltpu.sync_copy(dst_hbm.at[pl.ds(base, C)], dst_v)
        pltpu.async_copy(h_mine.at[src_v], rows_v, gsem).wait()

        def _grp(g, carry):
            sl = pl.ds(g * 16, 16)
            sv = src_v[sl]
            dv = dst_v[sl]
            e = plsc.load_gather(es_l, [sv]) + plsc.load_gather(ed_l, [dv])
            e = jnp.where(e >= 0.0, e, e * jnp.float32(0.2))
            ee = jnp.exp(e)
            ee_v[sl] = ee
            base = g * 16
            for t in range(16):
                eet = jnp.take_along_axis(
                    ee, jnp.full((16,), t, jnp.int32), axis=0,
                    mode="promise_in_bounds")
                r = base + t
                for j in range(DH // 16):
                    rs = pl.ds(j * 16, 16)
                    rows_v[r, rs] = rows_v[r, rs] * eet
            return carry
        lax.fori_loop(0, C // 16, _grp, 0)

        pltpu.sync_copy(rows_v, u_sh.at[dst_v], add=True)

        @pl.when(c == 0)
        def _():
            pltpu.sync_copy(ee_v, den_sh.at[dst_v], add=True)
        return carry

    lax.fori_loop(0, NCHUNK, _chunk, 0)

    plsc.subcore_barrier()

    pltpu.sync_copy(u_sh.at[pl.ds(row0, ROWS_PER_TILE)],
                    u_hbm.at[c, pl.ds(row0, ROWS_PER_TILE)])

    @pl.when(c == 0)
    def _():
        pltpu.sync_copy(den_sh.at[pl.ds(row0, ROWS_PER_TILE)],
                        den_hbm.at[pl.ds(row0, ROWS_PER_TILE)])


@functools.partial(
    pl.kernel,
    out_type=[
        jax.ShapeDtypeStruct((2, NPAD, DH), jnp.float32),
        jax.ShapeDtypeStruct((NPAD,), jnp.float32),
    ],
    mesh=plsc.VectorSubcoreMesh(core_axis_name="c", subcore_axis_name="s"),
    compiler_params=pltpu.CompilerParams(
        needs_layout_passes=False, use_tc_tiling_on_sc=False),
    scratch_types=[
        pltpu.VMEM((NPAD,), jnp.float32),
        pltpu.VMEM((NPAD,), jnp.float32),
        pltpu.VMEM((C,), jnp.int32),
        pltpu.VMEM((C,), jnp.int32),
        pltpu.VMEM((C, DH), jnp.float32),
        pltpu.VMEM((C,), jnp.float32),
        pltpu.VMEM((ROWS_PER_TILE,), jnp.float32),
        pltpu.VMEM_SHARED((NPAD, DH), jnp.float32),
        pltpu.VMEM_SHARED((NPAD,), jnp.float32),
        pltpu.SemaphoreType.DMA,
    ],
)
def _edge_pass(h_hbm, es_hbm, ed_hbm, src_hbm, dst_hbm, u_hbm, den_hbm,
               es_l, ed_l, src_v, dst_v, rows_v, ee_v, zden, u_sh, den_sh,
               gsem):
    _edge_body(h_hbm, es_hbm, ed_hbm, src_hbm, dst_hbm, u_hbm, den_hbm,
               es_l, ed_l, src_v, dst_v, rows_v, ee_v, zden, u_sh, den_sh,
               gsem)


def kernel(x, edge_index, W1, a_src1, a_dst1, W2, a_src2, a_dst2):
    x = x.astype(jnp.float32)
    x_pad = jnp.concatenate(
        [x, jnp.zeros((NPAD - N, D), jnp.float32)], axis=0)

    loop = jnp.arange(N, dtype=jnp.int32)
    padv = jnp.full((ETOT - E - N,), N, jnp.int32)
    src = jnp.concatenate([edge_index[0], loop, padv])
    dst = jnp.concatenate([edge_index[1], loop, padv])

    h1, es1, ed1 = _tc_in(x_pad, W1, a_src1.reshape(1, D), a_dst1.reshape(1, D))
    u1, den1 = _edge_pass(h1, es1.reshape(NPAD), ed1.reshape(NPAD), src, dst)

    h2, es2, ed2 = _tc_mid(u1, den1.reshape(NPAD, 1),
                           W2, a_src2.reshape(1, D), a_dst2.reshape(1, D))
    u2, den2 = _edge_pass(h2, es2.reshape(NPAD), ed2.reshape(NPAD), src, dst)

    return _tc_out(u2, den2.reshape(NPAD, 1))


# speedup vs baseline: 1.5145x; 1.5145x over previous
import functools

import jax
import jax.numpy as jnp
from jax import lax
from jax.experimental import pallas as pl
from jax.experimental.pallas import tpu as pltpu, tpu_sc as plsc

N = 10000
D = 128
DH = D // 2
E = 320000

C = 240
NCHUNK = 87
EPT = NCHUNK * C
ETOT = 16 * EPT
NPAD = 10240
ROWS_PER_TILE = NPAD // 16

_BLK = 512


def _tc_in_body(x_ref, w_ref, asrc_ref, adst_ref, h_ref, es_ref, ed_ref):
    h = jnp.dot(x_ref[...], w_ref[...], preferred_element_type=jnp.float32)
    h_ref[0] = h[:, :DH]
    h_ref[1] = h[:, DH:]
    es_ref[...] = jnp.sum(h * asrc_ref[...], axis=-1, keepdims=True)
    ed_ref[...] = jnp.sum(h * adst_ref[...], axis=-1, keepdims=True)


def _tc_in(x_pad, w, asrc, adst):
    grid = (NPAD // _BLK,)
    return pl.pallas_call(
        _tc_in_body,
        grid=grid,
        in_specs=[
            pl.BlockSpec((_BLK, D), lambda i: (i, 0)),
            pl.BlockSpec((D, D), lambda i: (0, 0)),
            pl.BlockSpec((1, D), lambda i: (0, 0)),
            pl.BlockSpec((1, D), lambda i: (0, 0)),
        ],
        out_specs=[
            pl.BlockSpec((2, _BLK, DH), lambda i: (0, i, 0)),
            pl.BlockSpec((_BLK, 1), lambda i: (i, 0)),
            pl.BlockSpec((_BLK, 1), lambda i: (i, 0)),
        ],
        out_shape=[
            jax.ShapeDtypeStruct((2, NPAD, DH), jnp.float32),
            jax.ShapeDtypeStruct((NPAD, 1), jnp.float32),
            jax.ShapeDtypeStruct((NPAD, 1), jnp.float32),
        ],
    )(x_pad, w, asrc, adst)


def _tc_mid_body(u_ref, d_ref, w_ref, asrc_ref, adst_ref,
                 h_ref, es_ref, ed_ref):
    den = jnp.maximum(d_ref[...], 1e-30)
    u = jnp.concatenate([u_ref[0], u_ref[1]], axis=-1)
    g = jnp.maximum(u, 0.0) / den
    h = jnp.dot(g, w_ref[...], preferred_element_type=jnp.float32)
    h_ref[0] = h[:, :DH]
    h_ref[1] = h[:, DH:]
    es_ref[...] = jnp.sum(h * asrc_ref[...], axis=-1, keepdims=True)
    ed_ref[...] = jnp.sum(h * adst_ref[...], axis=-1, keepdims=True)


def _tc_mid(u, d, w, asrc, adst):
    grid = (NPAD // _BLK,)
    return pl.pallas_call(
        _tc_mid_body,
        grid=grid,
        in_specs=[
            pl.BlockSpec((2, _BLK, DH), lambda i: (0, i, 0)),
            pl.BlockSpec((_BLK, 1), lambda i: (i, 0)),
            pl.BlockSpec((D, D), lambda i: (0, 0)),
            pl.BlockSpec((1, D), lambda i: (0, 0)),
            pl.BlockSpec((1, D), lambda i: (0, 0)),
        ],
        out_specs=[
            pl.BlockSpec((2, _BLK, DH), lambda i: (0, i, 0)),
            pl.BlockSpec((_BLK, 1), lambda i: (i, 0)),
            pl.BlockSpec((_BLK, 1), lambda i: (i, 0)),
        ],
        out_shape=[
            jax.ShapeDtypeStruct((2, NPAD, DH), jnp.float32),
            jax.ShapeDtypeStruct((NPAD, 1), jnp.float32),
            jax.ShapeDtypeStruct((NPAD, 1), jnp.float32),
        ],
    )(u, d, w, asrc, adst)


_OBLK = 400


def _tc_out_body(u_ref, d_ref, o_ref):
    den = jnp.maximum(d_ref[...], 1e-30)
    g = jnp.concatenate([u_ref[0], u_ref[1]], axis=-1) / den
    m = jnp.max(g, axis=-1, keepdims=True)
    l = g - m
    o_ref[...] = l - jnp.log(jnp.sum(jnp.exp(l), axis=-1, keepdims=True))


def _tc_out(u, d):
    grid = (N // _OBLK,)
    return pl.pallas_call(
        _tc_out_body,
        grid=grid,
        in_specs=[
            pl.BlockSpec((2, _OBLK, DH), lambda i: (0, i, 0)),
            pl.BlockSpec((_OBLK, 1), lambda i: (i, 0)),
        ],
        out_specs=pl.BlockSpec((_OBLK, D), lambda i: (i, 0)),
        out_shape=jax.ShapeDtypeStruct((N, D), jnp.float32),
    )(u, d)


def _edge_body(h_hbm, es_hbm, ed_hbm, src_hbm, dst_hbm,
               u_hbm, den_hbm,
               es_l, ed_l,
               src0, src1, src2, dst0, dst1, dst2,
               rows0, rows1, rows2, ee0, ee1, ee2, zden,
               u_sh, den_sh,
               g0, g1, g2, s0, s1, s2, d0, d1, d2, i0, i1, i2):
    c = lax.axis_index("c")
    s = lax.axis_index("s")

    src_bufs = (src0, src1, src2)
    dst_bufs = (dst0, dst1, dst2)
    rows_bufs = (rows0, rows1, rows2)
    ee_bufs = (ee0, ee1, ee2)
    gsems = (g0, g1, g2)
    ssems = (s0, s1, s2)
    dsems = (d0, d1, d2)
    isems = (i0, i1, i2)

    z16 = jnp.zeros((16,), jnp.float32)

    def _zrow(i, carry):
        for j in range(DH // 16):
            rows0[i, pl.ds(j * 16, 16)] = z16
        return carry
    lax.fori_loop(0, C, _zrow, 0)
    for t in range(ROWS_PER_TILE // 16):
        zden[pl.ds(t * 16, 16)] = z16

    row0 = pl.multiple_of(s * ROWS_PER_TILE, 8)
    pltpu.sync_copy(rows0, u_sh.at[pl.ds(row0, C)])
    pltpu.sync_copy(rows0, u_sh.at[pl.ds(row0 + C, C)])
    pltpu.sync_copy(rows0.at[pl.ds(0, ROWS_PER_TILE - 2 * C)],
                    u_sh.at[pl.ds(row0 + 2 * C, ROWS_PER_TILE - 2 * C)])
    pltpu.sync_copy(zden, den_sh.at[pl.ds(row0, ROWS_PER_TILE)])

    pltpu.sync_copy(es_hbm, es_l)
    pltpu.sync_copy(ed_hbm, ed_l)

    plsc.subcore_barrier()

    h_mine = h_hbm.at[c]
    ebase = pl.multiple_of(s * EPT, 8)

    def _idx_start(k, b):
        base = pl.multiple_of(ebase + k * C, 8)
        pltpu.async_copy(src_hbm.at[pl.ds(base, C)], src_bufs[b], isems[b])
        pltpu.async_copy(dst_hbm.at[pl.ds(base, C)], dst_bufs[b], isems[b])

    def _idx_wait(b):
        pltpu.make_async_copy(
            src_hbm.at[pl.ds(0, C)], src_bufs[b], isems[b]).wait()
        pltpu.make_async_copy(
            dst_hbm.at[pl.ds(0, C)], dst_bufs[b], isems[b]).wait()

    def _gather_start(b):
        pltpu.async_copy(h_mine.at[src_bufs[b]], rows_bufs[b], gsems[b])

    def _gather_wait(b):
        pltpu.make_async_copy(
            h_mine.at[src_bufs[b]], rows_bufs[b], gsems[b]).wait()

    def _scat_start(b):
        pltpu.async_copy(rows_bufs[b], u_sh.at[dst_bufs[b]], ssems[b],
                         add=True)
        pltpu.async_copy(ee_bufs[b], den_sh.at[dst_bufs[b]], dsems[b],
                         add=True)

    def _scat_wait(b):
        pltpu.make_async_copy(
            rows_bufs[b], u_sh.at[dst_bufs[b]], ssems[b]).wait()
        pltpu.make_async_copy(
            ee_bufs[b], den_sh.at[dst_bufs[b]], dsems[b]).wait()

    def _compute(b):
        src_b = src_bufs[b]
        dst_b = dst_bufs[b]
        rows_b = rows_bufs[b]
        ee_b = ee_bufs[b]
        for g in range(C // 16):
            sl = pl.ds(g * 16, 16)
            sv = src_b[sl]
            dv = dst_b[sl]
            e = plsc.load_gather(es_l, [sv]) + plsc.load_gather(ed_l, [dv])
            e = jnp.where(e >= 0.0, e, e * jnp.float32(0.2))
            ee_b[sl] = jnp.exp(e)

        def _scale(i, carry):
            eei = plsc.load_gather(ee_b, [jnp.full((16,), i, jnp.int32)])
            for j in range(DH // 16):
                sl = pl.ds(j * 16, 16)
                rows_b[i, sl] = rows_b[i, sl] * eei
            return carry
        lax.fori_loop(0, C, _scale, 0)

    _idx_start(0, 0)
    _idx_start(1, 1)
    _idx_wait(0)
    _gather_start(0)

    def _step(t, carry):
        for b in range(3):
            k = t * 3 + b
            nb = (b + 2) % 3

            @pl.when(k + 1 < NCHUNK)
            def _():
                _idx_wait((b + 1) % 3)
                _gather_start((b + 1) % 3)

            _gather_wait(b)
            _compute(b)
            _scat_start(b)

            @pl.when(k >= 1)
            def _():
                _scat_wait(nb)

            @pl.when(k + 2 < NCHUNK)
            def _():
                _idx_start(k + 2, nb)
        return carry

    lax.fori_loop(0, NCHUNK // 3, _step, 0)

    _scat_wait((NCHUNK - 1) % 3)

    plsc.subcore_barrier()

    pltpu.sync_copy(u_sh.at[pl.ds(row0, ROWS_PER_TILE)],
                    u_hbm.at[c, pl.ds(row0, ROWS_PER_TILE)])

    @pl.when(c == 0)
    def _():
        pltpu.sync_copy(den_sh.at[pl.ds(row0, ROWS_PER_TILE)],
                        den_hbm.at[pl.ds(row0, ROWS_PER_TILE)])


@functools.partial(
    pl.kernel,
    out_type=[
        jax.ShapeDtypeStruct((2, NPAD, DH), jnp.float32),
        jax.ShapeDtypeStruct((NPAD,), jnp.float32),
    ],
    mesh=plsc.VectorSubcoreMesh(core_axis_name="c", subcore_axis_name="s"),
    compiler_params=pltpu.CompilerParams(
        needs_layout_passes=False, use_tc_tiling_on_sc=False),
    scratch_types=[
        pltpu.VMEM((NPAD,), jnp.float32),
        pltpu.VMEM((NPAD,), jnp.float32),
        pltpu.VMEM((C,), jnp.int32),
        pltpu.VMEM((C,), jnp.int32),
        pltpu.VMEM((C,), jnp.int32),
        pltpu.VMEM((C,), jnp.int32),
        pltpu.VMEM((C,), jnp.int32),
        pltpu.VMEM((C,), jnp.int32),
        pltpu.VMEM((C, DH), jnp.float32),
        pltpu.VMEM((C, DH), jnp.float32),
        pltpu.VMEM((C, DH), jnp.float32),
        pltpu.VMEM((C,), jnp.float32),
        pltpu.VMEM((C,), jnp.float32),
        pltpu.VMEM((C,), jnp.float32),
        pltpu.VMEM((ROWS_PER_TILE,), jnp.float32),
        pltpu.VMEM_SHARED((NPAD, DH), jnp.float32),
        pltpu.VMEM_SHARED((NPAD,), jnp.float32),
        pltpu.SemaphoreType.DMA,
        pltpu.SemaphoreType.DMA,
        pltpu.SemaphoreType.DMA,
        pltpu.SemaphoreType.DMA,
        pltpu.SemaphoreType.DMA,
        pltpu.SemaphoreType.DMA,
        pltpu.SemaphoreType.DMA,
        pltpu.SemaphoreType.DMA,
        pltpu.SemaphoreType.DMA,
        pltpu.SemaphoreType.DMA,
        pltpu.SemaphoreType.DMA,
        pltpu.SemaphoreType.DMA,
    ],
)
def _edge_pass(h_hbm, es_hbm, ed_hbm, src_hbm, dst_hbm, u_hbm, den_hbm,
               es_l, ed_l, src0, src1, src2, dst0, dst1, dst2,
               rows0, rows1, rows2, ee0, ee1, ee2, zden, u_sh, den_sh,
               g0, g1, g2, s0, s1, s2, d0, d1, d2, i0, i1, i2):
    _edge_body(h_hbm, es_hbm, ed_hbm, src_hbm, dst_hbm, u_hbm, den_hbm,
               es_l, ed_l, src0, src1, src2, dst0, dst1, dst2,
               rows0, rows1, rows2, ee0, ee1, ee2, zden, u_sh, den_sh,
               g0, g1, g2, s0, s1, s2, d0, d1, d2, i0, i1, i2)


def kernel(x, edge_index, W1, a_src1, a_dst1, W2, a_src2, a_dst2):
    x = x.astype(jnp.float32)
    x_pad = jnp.concatenate(
        [x, jnp.zeros((NPAD - N, D), jnp.float32)], axis=0)

    loop = jnp.arange(N, dtype=jnp.int32)
    padv = jnp.full((ETOT - E - N,), N, jnp.int32)
    src = jnp.concatenate([edge_index[0], loop, padv])
    dst = jnp.concatenate([edge_index[1], loop, padv])

    h1, es1, ed1 = _tc_in(x_pad, W1, a_src1.reshape(1, D), a_dst1.reshape(1, D))
    u1, den1 = _edge_pass(h1, es1.reshape(NPAD), ed1.reshape(NPAD), src, dst)

    h2, es2, ed2 = _tc_mid(u1, den1.reshape(NPAD, 1),
                           W2, a_src2.reshape(1, D), a_dst2.reshape(1, D))
    u2, den2 = _edge_pass(h2, es2.reshape(NPAD), ed2.reshape(NPAD), src, dst)

    return _tc_out(u2, den2.reshape(NPAD, 1))
